# baseline (speedup 1.0000x reference)
import functools

import jax
import jax.numpy as jnp
from jax import lax
from jax.experimental import pallas as pl
from jax.experimental.pallas import tpu as pltpu
from jax.experimental.pallas import tpu_sc as plsc

VOCAB = 1000000
EMB = 64
W = 128
B = 16384
NEG = 20

NC = 2
NS = 16
NW = NC * NS
BPW = B // NW
GR = 128
NCH = BPW * NEG // GR

DCH = 160
NDCH = VOCAB // DCH
DPW = (NDCH + NW - 1) // NW


def _sc_pad(in_hbm, out_hbm, in128, out128,
            a64, a128, b64, b128, semai, sembi, semao, sembo):
    wid = lax.axis_index("s") * NC + lax.axis_index("c")

    def chunk_of(t):
        return wid + t * NW

    @pl.when(chunk_of(0) < NDCH)
    def _():
        r0 = chunk_of(0) * DCH
        pltpu.async_copy(in_hbm.at[pl.ds(r0, DCH)], a64, semai)
        pltpu.async_copy(out_hbm.at[pl.ds(r0, DCH)], b64, sembi)

    def widen(src, dst):
        def vrow(i, carry):
            for k in range(EMB // 16):
                dst[i, pl.ds(k * 16, 16)] = src[i, pl.ds(k * 16, 16)]
            return carry

        lax.fori_loop(0, DCH, vrow, 0)

    def body(t, carry):
        g = chunk_of(t)

        @pl.when(g < NDCH)
        def _():
            r0 = g * DCH

            @pl.when(t > 0)
            def _():
                pltpu.make_async_copy(a128, in128.at[pl.ds(0, DCH)],
                                      semao).wait()
                pltpu.make_async_copy(b128, out128.at[pl.ds(0, DCH)],
                                      sembo).wait()

            pltpu.make_async_copy(in_hbm.at[pl.ds(0, DCH)], a64, semai).wait()
            widen(a64, a128)
            pltpu.async_copy(a128, in128.at[pl.ds(r0, DCH)], semao)
            pltpu.make_async_copy(out_hbm.at[pl.ds(0, DCH)], b64, sembi).wait()
            widen(b64, b128)
            pltpu.async_copy(b128, out128.at[pl.ds(r0, DCH)], sembo)

            @pl.when(g + NW < NDCH)
            def _():
                rn = (g + NW) * DCH
                pltpu.async_copy(in_hbm.at[pl.ds(rn, DCH)], a64, semai)
                pltpu.async_copy(out_hbm.at[pl.ds(rn, DCH)], b64, sembi)

        return carry

    lax.fori_loop(0, DPW, body, 0)

    @pl.when(chunk_of(0) < NDCH)
    def _():
        pltpu.make_async_copy(a128, in128.at[pl.ds(0, DCH)], semao).wait()
        pltpu.make_async_copy(b128, out128.at[pl.ds(0, DCH)], sembo).wait()


def _sc_rows(in_hbm, out_hbm, tgt_hbm, ctx_hbm, t_out, c_out,
             tidx_v, cidx_v, rows_v, tsem, csem):
    wid = lax.axis_index("s") * NC + lax.axis_index("c")
    base = wid * BPW
    pltpu.sync_copy(tgt_hbm.at[pl.ds(wid * (BPW // GR), BPW // GR)], tidx_v)
    pltpu.sync_copy(ctx_hbm.at[pl.ds(wid * (BPW // GR), BPW // GR)], cidx_v)
    tcp = [pltpu.async_copy(in_hbm.at[tidx_v.at[j]],
                            rows_v.at[pl.ds(j * GR, GR)], tsem)
           for j in range(BPW // GR)]
    for c in tcp:
        c.wait()
    pltpu.sync_copy(rows_v, t_out.at[pl.ds(base, BPW)])
    ccp = [pltpu.async_copy(out_hbm.at[cidx_v.at[j]],
                            rows_v.at[pl.ds(j * GR, GR)], csem)
           for j in range(BPW // GR)]
    for c in ccp:
        c.wait()
    pltpu.sync_copy(rows_v, c_out.at[pl.ds(base, BPW)])


def _sc_neg(out_hbm, neg_hbm, scat_hbm, zer_hbm, n_out,
            acc_sh, nidx_v, sidx_v, nbuf0_v, nbuf1_v, sem0, sem1, ssem):
    sid = lax.axis_index("s")
    wid = sid * NC + lax.axis_index("c")
    base = wid * BPW

    pltpu.sync_copy(zer_hbm, acc_sh.at[pl.ds(sid * BPW, BPW)])
    pltpu.sync_copy(neg_hbm.at[pl.ds(wid * NCH, NCH)], nidx_v)
    pltpu.sync_copy(scat_hbm.at[pl.ds(wid * NCH, NCH)], sidx_v)
    plsc.subcore_barrier()

    pltpu.async_copy(out_hbm.at[nidx_v.at[0]], nbuf0_v, sem0)

    def body(i, carry):
        j = 2 * i
        pltpu.make_async_copy(out_hbm.at[nidx_v.at[j]], nbuf0_v, sem0).wait()
        pltpu.async_copy(out_hbm.at[nidx_v.at[j + 1]], nbuf1_v, sem1)
        pltpu.async_copy(nbuf0_v, acc_sh.at[sidx_v.at[j]], ssem,
                         add=True).wait()
        pltpu.make_async_copy(out_hbm.at[nidx_v.at[j + 1]], nbuf1_v,
                              sem1).wait()

        @pl.when(i < NCH // 2 - 1)
        def _():
            pltpu.async_copy(out_hbm.at[nidx_v.at[j + 2]], nbuf0_v, sem0)

        pltpu.async_copy(nbuf1_v, acc_sh.at[sidx_v.at[j + 1]], ssem,
                         add=True).wait()
        return carry

    lax.fori_loop(0, NCH // 2, body, 0)
    plsc.subcore_barrier()
    pltpu.sync_copy(acc_sh.at[pl.ds(sid * BPW, BPW)], n_out.at[pl.ds(base, BPW)])


def _tc_reduce(t_ref, c_ref, n_ref, o_ref):
    t = t_ref[:, :EMB]
    score = jnp.sum(t * c_ref[:, :EMB], axis=1)
    neg = jnp.sum(t * n_ref[:, :EMB], axis=1)
    loss = -(jnp.sum(jax.nn.log_sigmoid(score))
             + jnp.sum(jax.nn.log_sigmoid(-neg)))
    o_ref[...] = jnp.reshape(loss, (1, 1))


def kernel(in_embed, out_embed, target, context, neg_context):
    f32 = jnp.float32
    i32 = jnp.int32
    mesh = plsc.VectorSubcoreMesh(core_axis_name="c", subcore_axis_name="s",
                                  num_cores=NC)
    scp = pltpu.CompilerParams(use_tc_tiling_on_sc=True)

    pad_fn = functools.partial(
        pl.kernel,
        mesh=mesh,
        compiler_params=scp,
        out_type=[jax.ShapeDtypeStruct((VOCAB, W), f32)] * 2,
        scratch_types=[
            pltpu.VMEM((DCH, EMB), f32),
            pltpu.VMEM((DCH, W), f32),
            pltpu.VMEM((DCH, EMB), f32),
            pltpu.VMEM((DCH, W), f32),
            pltpu.SemaphoreType.DMA,
            pltpu.SemaphoreType.DMA,
            pltpu.SemaphoreType.DMA,
            pltpu.SemaphoreType.DMA,
        ],
    )(_sc_pad)
    in128, out128 = pad_fn(in_embed, out_embed)

    tgt2 = target.astype(i32).reshape(B // GR, GR)
    ctx2 = context.astype(i32).reshape(B // GR, GR)

    rows_fn = functools.partial(
        pl.kernel,
        mesh=mesh,
        compiler_params=scp,
        out_type=[jax.ShapeDtypeStruct((B, W), f32)] * 2,
        scratch_types=[
            pltpu.VMEM((BPW // GR, GR), i32),
            pltpu.VMEM((BPW // GR, GR), i32),
            pltpu.VMEM((BPW, W), f32),
            pltpu.SemaphoreType.DMA,
            pltpu.SemaphoreType.DMA,
        ],
    )(_sc_rows)
    t_rows, c_rows = rows_fn(in128, out128, tgt2, ctx2)

    neg2 = neg_context.astype(i32).reshape(B * NEG // GR, GR)
    local = jnp.repeat(jnp.arange(BPW, dtype=i32), NEG)
    scat2 = ((jnp.arange(NW, dtype=i32) // NC * BPW)[:, None]
             + local[None, :]).reshape(B * NEG // GR, GR)
    zeros = jnp.zeros((BPW, W), f32)

    neg_fn = functools.partial(
        pl.kernel,
        mesh=mesh,
        compiler_params=scp,
        out_type=jax.ShapeDtypeStruct((B, W), f32),
        scratch_types=[
            pltpu.VMEM_SHARED((NS * BPW, W), f32),
            pltpu.VMEM((NCH, GR), i32),
            pltpu.VMEM((NCH, GR), i32),
            pltpu.VMEM((GR, W), f32),
            pltpu.VMEM((GR, W), f32),
            pltpu.SemaphoreType.DMA,
            pltpu.SemaphoreType.DMA,
            pltpu.SemaphoreType.DMA,
        ],
    )(_sc_neg)
    n_sum = neg_fn(out128, neg2, scat2, zeros)

    loss = pl.pallas_call(
        _tc_reduce,
        out_shape=jax.ShapeDtypeStruct((1, 1), f32),
    )(t_rows, c_rows, n_sum)
    return loss[0, 0]


# speedup vs baseline: 1.2694x; 1.2694x over previous
import functools

import jax
import jax.numpy as jnp
from jax import lax
from jax.experimental import pallas as pl
from jax.experimental.pallas import tpu as pltpu
from jax.experimental.pallas import tpu_sc as plsc

VOCAB = 1000000
EMB = 64
B = 16384
NEG = 20

NC = 2
NS = 16
NW = NC * NS
BPW = B // NW
GR = 128
NCH = BPW * NEG // GR


def _sc_gather(in_hbm, out_hbm, tgt_hbm, ctx_hbm, neg_hbm, scat_hbm, zer_hbm,
               t_out, c_out, n_out,
               idx_v, cidx_v, rows_v, acc_sh, nidx_v, sidx_v,
               nbuf0_v, nbuf1_v, tsem, sem0, sem1, ssem):
    sid = lax.axis_index("s")
    wid = sid * NC + lax.axis_index("c")
    base = wid * BPW

    pltpu.sync_copy(tgt_hbm.at[pl.ds(wid * (BPW // GR), BPW // GR)], idx_v)
    pltpu.sync_copy(ctx_hbm.at[pl.ds(wid * (BPW // GR), BPW // GR)], cidx_v)
    tcp = [pltpu.async_copy(in_hbm.at[idx_v.at[j]],
                            rows_v.at[pl.ds(j * GR, GR)], tsem)
           for j in range(BPW // GR)]

    pltpu.sync_copy(zer_hbm, acc_sh.at[pl.ds(sid * BPW, BPW)])
    pltpu.sync_copy(neg_hbm.at[pl.ds(wid * NCH, NCH)], nidx_v)
    pltpu.sync_copy(scat_hbm.at[pl.ds(wid * NCH, NCH)], sidx_v)
    plsc.subcore_barrier()

    def neg_span(lo, hi):
        pltpu.async_copy(out_hbm.at[nidx_v.at[lo]], nbuf0_v, sem0)

        def body(i, carry):
            j = 2 * i
            pltpu.make_async_copy(out_hbm.at[nidx_v.at[j]], nbuf0_v,
                                  sem0).wait()
            pltpu.async_copy(out_hbm.at[nidx_v.at[j + 1]], nbuf1_v, sem1)
            pltpu.async_copy(nbuf0_v, acc_sh.at[sidx_v.at[j]], ssem,
                             add=True).wait()
            pltpu.make_async_copy(out_hbm.at[nidx_v.at[j + 1]], nbuf1_v,
                                  sem1).wait()

            @pl.when(i < hi // 2 - 1)
            def _():
                pltpu.async_copy(out_hbm.at[nidx_v.at[j + 2]], nbuf0_v, sem0)

            pltpu.async_copy(nbuf1_v, acc_sh.at[sidx_v.at[j + 1]], ssem,
                             add=True).wait()
            return carry

        lax.fori_loop(lo // 2, hi // 2, body, 0)

    neg_span(0, NCH // 2)
    for c in tcp:
        c.wait()
    pltpu.sync_copy(rows_v, t_out.at[pl.ds(base, BPW)])
    ccp = [pltpu.async_copy(out_hbm.at[cidx_v.at[j]],
                            rows_v.at[pl.ds(j * GR, GR)], tsem)
           for j in range(BPW // GR)]
    neg_span(NCH // 2, NCH)

    plsc.subcore_barrier()
    pltpu.sync_copy(acc_sh.at[pl.ds(sid * BPW, BPW)], n_out.at[pl.ds(base, BPW)])
    for c in ccp:
        c.wait()
    pltpu.sync_copy(rows_v, c_out.at[pl.ds(base, BPW)])


def _tc_reduce(t_ref, c_ref, n_ref, o_ref):
    t = t_ref[...]
    score = jnp.sum(t * c_ref[...], axis=1)
    neg = jnp.sum(t * n_ref[...], axis=1)
    loss = -(jnp.sum(jax.nn.log_sigmoid(score))
             + jnp.sum(jax.nn.log_sigmoid(-neg)))
    o_ref[...] = jnp.reshape(loss, (1, 1))


def kernel(in_embed, out_embed, target, context, neg_context):
    f32 = jnp.float32
    tgt2 = target.astype(jnp.int32).reshape(B // GR, GR)
    ctx2 = context.astype(jnp.int32).reshape(B // GR, GR)
    neg2 = neg_context.astype(jnp.int32).reshape(B * NEG // GR, GR)
    local = jnp.repeat(jnp.arange(BPW, dtype=jnp.int32), NEG)
    scat2 = ((jnp.arange(NW, dtype=jnp.int32) // NC * BPW)[:, None]
             + local[None, :]).reshape(B * NEG // GR, GR)
    zeros = jnp.zeros((BPW, EMB), f32)

    sc_fn = functools.partial(
        pl.kernel,
        mesh=plsc.VectorSubcoreMesh(core_axis_name="c", subcore_axis_name="s",
                                    num_cores=NC),
        compiler_params=pltpu.CompilerParams(use_tc_tiling_on_sc=False),
        out_type=[jax.ShapeDtypeStruct((B, EMB), f32)] * 3,
        scratch_types=[
            pltpu.VMEM((BPW // GR, GR), jnp.int32),
            pltpu.VMEM((BPW // GR, GR), jnp.int32),
            pltpu.VMEM((BPW, EMB), f32),
            pltpu.VMEM_SHARED((NS * BPW, EMB), f32),
            pltpu.VMEM((NCH, GR), jnp.int32),
            pltpu.VMEM((NCH, GR), jnp.int32),
            pltpu.VMEM((GR, EMB), f32),
            pltpu.VMEM((GR, EMB), f32),
            pltpu.SemaphoreType.DMA,
            pltpu.SemaphoreType.DMA,
            pltpu.SemaphoreType.DMA,
            pltpu.SemaphoreType.DMA,
        ],
    )(_sc_gather)

    t_rows, c_rows, n_sum = sc_fn(in_embed, out_embed, tgt2, ctx2, neg2,
                                  scat2, zeros)

    loss = pl.pallas_call(
        _tc_reduce,
        out_shape=jax.ShapeDtypeStruct((1, 1), f32),
    )(t_rows, c_rows, n_sum)
    return loss[0, 0]
